# baseline (speedup 1.0000x reference)
import functools

import jax
import jax.numpy as jnp
from jax import lax
from jax.experimental import pallas as pl
from jax.experimental.pallas import tpu as pltpu
from jax.experimental.pallas import tpu_sc as plsc

B = 1024
NUM_ENT = 1000
DIM = 64
D_PAD = 128
N_PAD = 1024
B_BLK = 128
CH = 128
BS = 32
EPS = 1e-12


def _sc_gather(ent_emb, rel_emb, e1, rel):
    info = plsc.get_sparse_core_info()
    nw = info.num_cores * info.num_subcores
    b_per_w = B // nw
    mesh = plsc.VectorSubcoreMesh(core_axis_name="c", subcore_axis_name="s")

    @functools.partial(
        pl.kernel,
        mesh=mesh,
        compiler_params=pltpu.CompilerParams(use_tc_tiling_on_sc=False),
        out_type=[
            jax.ShapeDtypeStruct((B, DIM), jnp.float32),
            jax.ShapeDtypeStruct((B, DIM), jnp.float32),
        ],
        scratch_types=[
            pltpu.VMEM((b_per_w,), jnp.int32),
            pltpu.VMEM((b_per_w,), jnp.int32),
            pltpu.VMEM((b_per_w, DIM), jnp.float32),
            pltpu.VMEM((b_per_w, DIM), jnp.float32),
            pltpu.SemaphoreType.DMA,
            pltpu.SemaphoreType.DMA,
        ],
    )
    def gk(ent_hbm, rel_hbm, e1_hbm, ridx_hbm, oute_hbm, outr_hbm,
           idx1_v, idx2_v, rows1_v, rows2_v, sem1, sem2):
        wid = lax.axis_index("s") * info.num_cores + lax.axis_index("c")
        base = wid * b_per_w
        ci1 = pltpu.async_copy(e1_hbm.at[pl.ds(base, b_per_w)], idx1_v, sem1)
        ci2 = pltpu.async_copy(ridx_hbm.at[pl.ds(base, b_per_w)], idx2_v, sem2)
        ci1.wait()
        c1 = pltpu.async_copy(ent_hbm.at[idx1_v], rows1_v, sem1)
        ci2.wait()
        c2 = pltpu.async_copy(rel_hbm.at[idx2_v], rows2_v, sem2)
        c1.wait()
        co1 = pltpu.async_copy(rows1_v, oute_hbm.at[pl.ds(base, b_per_w)], sem1)
        c2.wait()
        co2 = pltpu.async_copy(rows2_v, outr_hbm.at[pl.ds(base, b_per_w)], sem2)
        co1.wait()
        co2.wait()

    return gk(ent_emb, rel_emb, e1, rel)


def _rnorm(x):
    n = jnp.sqrt(jnp.sum(x * x, axis=-1, keepdims=True))
    return x / jnp.maximum(n, EPS)


def _score_body(e1r_ref, relr_ref, ent_ref, out_ref, entn_ref, sn_ref,
                hb_ref):
    @pl.when(pl.program_id(0) == 0)
    def _():
        ent_t = jnp.transpose(_rnorm(ent_ref[...]))
        ent_t = jnp.concatenate(
            [ent_t, jnp.zeros((DIM, N_PAD - NUM_ENT), jnp.float32)], axis=1)
        sn_ref[...] = jnp.sum(ent_t, axis=0, keepdims=True)
        entn_ref[...] = ent_t + ent_t

    h2 = _rnorm(e1r_ref[...]) + _rnorm(relr_ref[...])
    h2 = h2 + h2
    ent_t2 = entn_ref[...]
    sn = sn_ref[...]

    lane = lax.broadcasted_iota(jnp.int32, (1, N_PAD), 1)
    valid = lane < NUM_ENT

    for d in range(DIM):
        hb_ref[d] = jnp.broadcast_to(h2[:, d:d + 1], (B_BLK, CH))
    chunks = []
    for cp in range(N_PAD // (2 * CH)):
        sl0 = ent_t2[:, (2 * cp) * CH:(2 * cp + 1) * CH]
        sl1 = ent_t2[:, (2 * cp + 1) * CH:(2 * cp + 2) * CH]
        hv = hb_ref[0]
        acc0 = jnp.minimum(hv, sl0[0:1, :])
        acc1 = jnp.minimum(hv, sl1[0:1, :])
        for d in range(1, DIM):
            hv = hb_ref[d]
            acc0 = acc0 + jnp.minimum(hv, sl0[d:d + 1, :])
            acc1 = acc1 + jnp.minimum(hv, sl1[d:d + 1, :])
        chunks.append(sn[:, (2 * cp) * CH:(2 * cp + 1) * CH] - acc0)
        chunks.append(sn[:, (2 * cp + 1) * CH:(2 * cp + 2) * CH] - acc1)
    dist = jnp.concatenate(chunks, axis=1)
    dist = jnp.where(valid, dist, -jnp.inf)
    m = jnp.max(dist, axis=-1, keepdims=True)
    e = jnp.exp(dist - m)
    e = jnp.where(valid, e, 0.0)
    s = jnp.sum(e, axis=-1, keepdims=True)
    out_ref[...] = (e / s)[:, :NUM_ENT]


def kernel(e1, rel, X, A, ent_emb, rel_emb):
    del X, A
    e1 = e1.astype(jnp.int32)
    rel = rel.astype(jnp.int32)
    e1_rows, rel_rows = _sc_gather(ent_emb, rel_emb, e1, rel)
    return pl.pallas_call(
        _score_body,
        grid=(B // B_BLK,),
        in_specs=[
            pl.BlockSpec((B_BLK, DIM), lambda i: (i, 0)),
            pl.BlockSpec((B_BLK, DIM), lambda i: (i, 0)),
            pl.BlockSpec((NUM_ENT, DIM), lambda i: (0, 0)),
        ],
        out_specs=pl.BlockSpec((B_BLK, NUM_ENT), lambda i: (i, 0)),
        out_shape=jax.ShapeDtypeStruct((B, NUM_ENT), jnp.float32),
        scratch_shapes=[
            pltpu.VMEM((DIM, N_PAD), jnp.float32),
            pltpu.VMEM((1, N_PAD), jnp.float32),
            pltpu.VMEM((DIM, B_BLK, CH), jnp.float32),
        ],
    )(e1_rows, rel_rows, ent_emb)


# speedup vs baseline: 1.2739x; 1.2739x over previous
import functools

import jax
import jax.numpy as jnp
from jax import lax
from jax.experimental import pallas as pl
from jax.experimental.pallas import tpu as pltpu
from jax.experimental.pallas import tpu_sc as plsc

B = 1024
NUM_ENT = 1000
DIM = 64
N_PAD = 1024
B_BLK = 128
CH = 128
EPS = 1e-12


def _sc_gather(ent_emb, rel_emb, e1, rel):
    info = plsc.get_sparse_core_info()
    nw = info.num_cores * info.num_subcores
    b_per_w = B // nw
    mesh = plsc.VectorSubcoreMesh(core_axis_name="c", subcore_axis_name="s")

    @functools.partial(
        pl.kernel,
        mesh=mesh,
        compiler_params=pltpu.CompilerParams(use_tc_tiling_on_sc=False),
        out_type=[
            jax.ShapeDtypeStruct((B, DIM), jnp.float32),
            jax.ShapeDtypeStruct((B, DIM), jnp.float32),
        ],
        scratch_types=[
            pltpu.VMEM((b_per_w,), jnp.int32),
            pltpu.VMEM((b_per_w,), jnp.int32),
            pltpu.VMEM((b_per_w, DIM), jnp.float32),
            pltpu.VMEM((b_per_w, DIM), jnp.float32),
            pltpu.SemaphoreType.DMA,
            pltpu.SemaphoreType.DMA,
        ],
    )
    def gk(ent_hbm, rel_hbm, e1_hbm, ridx_hbm, oute_hbm, outr_hbm,
           idx1_v, idx2_v, rows1_v, rows2_v, sem1, sem2):
        wid = lax.axis_index("s") * info.num_cores + lax.axis_index("c")
        base = wid * b_per_w
        ci1 = pltpu.async_copy(e1_hbm.at[pl.ds(base, b_per_w)], idx1_v, sem1)
        ci2 = pltpu.async_copy(ridx_hbm.at[pl.ds(base, b_per_w)], idx2_v, sem2)
        ci1.wait()
        c1 = pltpu.async_copy(ent_hbm.at[idx1_v], rows1_v, sem1)
        ci2.wait()
        c2 = pltpu.async_copy(rel_hbm.at[idx2_v], rows2_v, sem2)
        c1.wait()
        co1 = pltpu.async_copy(rows1_v, oute_hbm.at[pl.ds(base, b_per_w)], sem1)
        c2.wait()
        co2 = pltpu.async_copy(rows2_v, outr_hbm.at[pl.ds(base, b_per_w)], sem2)
        co1.wait()
        co2.wait()

    return gk(ent_emb, rel_emb, e1, rel)


def _rnorm(x):
    n = jnp.sqrt(jnp.sum(x * x, axis=-1, keepdims=True))
    return x / jnp.maximum(n, EPS)


def _score_body(e1r_ref, relr_ref, entT_ref, out_ref):
    h = _rnorm(e1r_ref[...]) + _rnorm(relr_ref[...])

    ent_t = entT_ref[...]
    n = jnp.sqrt(jnp.sum(ent_t * ent_t, axis=0, keepdims=True))
    ent_tn = ent_t / jnp.maximum(n, EPS)
    sn = jnp.sum(ent_tn, axis=0, keepdims=True)

    hb = [jnp.broadcast_to(h[:, d:d + 1], (B_BLK, CH)) for d in range(DIM)]

    chunks = []
    for c in range(N_PAD // CH):
        sl = ent_tn[:, c * CH:(c + 1) * CH]
        acc = jnp.minimum(hb[0], sl[0:1, :])
        for d in range(1, DIM):
            acc = acc + jnp.minimum(hb[d], sl[d:d + 1, :])
        chunks.append(sn[:, c * CH:(c + 1) * CH] - (acc + acc))
    dist = jnp.concatenate(chunks, axis=1)

    lane = lax.broadcasted_iota(jnp.int32, (1, N_PAD), 1)
    valid = lane < NUM_ENT
    dist = jnp.where(valid, dist, -jnp.inf)
    m = jnp.max(dist, axis=-1, keepdims=True)
    e = jnp.exp(dist - m)
    e = jnp.where(valid, e, 0.0)
    s = jnp.sum(e, axis=-1, keepdims=True)
    out_ref[...] = (e / s)[:, :NUM_ENT]


def kernel(e1, rel, X, A, ent_emb, rel_emb):
    del X, A
    e1 = e1.astype(jnp.int32)
    rel = rel.astype(jnp.int32)
    e1_rows, rel_rows = _sc_gather(ent_emb, rel_emb, e1, rel)
    ent_t = jnp.pad(ent_emb.T, ((0, 0), (0, N_PAD - NUM_ENT)))
    return pl.pallas_call(
        _score_body,
        grid=(B // B_BLK,),
        in_specs=[
            pl.BlockSpec((B_BLK, DIM), lambda i: (i, 0)),
            pl.BlockSpec((B_BLK, DIM), lambda i: (i, 0)),
            pl.BlockSpec((DIM, N_PAD), lambda i: (0, 0)),
        ],
        out_specs=pl.BlockSpec((B_BLK, NUM_ENT), lambda i: (i, 0)),
        out_shape=jax.ShapeDtypeStruct((B, NUM_ENT), jnp.float32),
    )(e1_rows, rel_rows, ent_t)


# speedup vs baseline: 1.3600x; 1.0676x over previous
import functools

import jax
import jax.numpy as jnp
from jax import lax
from jax.experimental import pallas as pl
from jax.experimental.pallas import tpu as pltpu
from jax.experimental.pallas import tpu_sc as plsc

B = 1024
NUM_ENT = 1000
DIM = 64
D_PAD = 128
N_PAD = 1024
B_BLK = 128
CH = 128
EPS = 1e-12


def _sc_gather(ent_emb, rel_emb, e1, rel):
    info = plsc.get_sparse_core_info()
    nw = info.num_cores * info.num_subcores
    b_per_w = B // nw
    mesh = plsc.VectorSubcoreMesh(core_axis_name="c", subcore_axis_name="s")

    @functools.partial(
        pl.kernel,
        mesh=mesh,
        out_type=[
            jax.ShapeDtypeStruct((B, D_PAD), jnp.float32),
            jax.ShapeDtypeStruct((B, D_PAD), jnp.float32),
        ],
        scratch_types=[
            pltpu.VMEM((b_per_w,), jnp.int32),
            pltpu.VMEM((b_per_w,), jnp.int32),
            pltpu.VMEM((b_per_w, D_PAD), jnp.float32),
            pltpu.VMEM((b_per_w, D_PAD), jnp.float32),
            pltpu.SemaphoreType.DMA,
            pltpu.SemaphoreType.DMA,
        ],
    )
    def gk(ent_hbm, rel_hbm, e1_hbm, ridx_hbm, oute_hbm, outr_hbm,
           idx1_v, idx2_v, rows1_v, rows2_v, sem1, sem2):
        wid = lax.axis_index("s") * info.num_cores + lax.axis_index("c")
        base = wid * b_per_w
        ci1 = pltpu.async_copy(e1_hbm.at[pl.ds(base, b_per_w)], idx1_v, sem1)
        ci2 = pltpu.async_copy(ridx_hbm.at[pl.ds(base, b_per_w)], idx2_v, sem2)
        ci1.wait()
        c1 = pltpu.async_copy(ent_hbm.at[idx1_v], rows1_v, sem1)
        ci2.wait()
        c2 = pltpu.async_copy(rel_hbm.at[idx2_v], rows2_v, sem2)
        c1.wait()
        co1 = pltpu.async_copy(rows1_v, oute_hbm.at[pl.ds(base, b_per_w)], sem1)
        c2.wait()
        co2 = pltpu.async_copy(rows2_v, outr_hbm.at[pl.ds(base, b_per_w)], sem2)
        co1.wait()
        co2.wait()

    return gk(ent_emb, rel_emb, e1, rel)


def _rnorm(x):
    n = jnp.sqrt(jnp.sum(x * x, axis=-1, keepdims=True))
    return x / jnp.maximum(n, EPS)


def _score_body(e1r_ref, relr_ref, entT_ref, out_ref):
    h = _rnorm(e1r_ref[:, :DIM]) + _rnorm(relr_ref[:, :DIM])

    ent_t = entT_ref[...]
    n = jnp.sqrt(jnp.sum(ent_t * ent_t, axis=0, keepdims=True))
    ent_tn = ent_t / jnp.maximum(n, EPS)
    lane = lax.broadcasted_iota(jnp.int32, (1, N_PAD), 1)
    sn = jnp.where(lane < NUM_ENT,
                   jnp.sum(ent_tn, axis=0, keepdims=True),
                   -jnp.inf)

    hb = [jnp.broadcast_to(h[:, d:d + 1], (B_BLK, CH)) for d in range(DIM)]

    chunks = []
    for c in range(N_PAD // CH):
        sl = ent_tn[:, c * CH:(c + 1) * CH]
        acc = jnp.minimum(hb[0], sl[0:1, :])
        for d in range(1, DIM):
            acc = acc + jnp.minimum(hb[d], sl[d:d + 1, :])
        chunks.append(sn[:, c * CH:(c + 1) * CH] - (acc + acc))
    dist = jnp.concatenate(chunks, axis=1)

    m = jnp.max(dist, axis=-1, keepdims=True)
    e = jnp.exp(dist - m)
    s = jnp.sum(e, axis=-1, keepdims=True)
    out_ref[...] = (e / s)[:, :NUM_ENT]


def kernel(e1, rel, X, A, ent_emb, rel_emb):
    del X, A
    e1 = e1.astype(jnp.int32)
    rel = rel.astype(jnp.int32)
    ent_pad = jnp.pad(ent_emb, ((0, 0), (0, D_PAD - DIM)))
    rel_pad = jnp.pad(rel_emb, ((0, 0), (0, D_PAD - DIM)))
    e1_rows, rel_rows = _sc_gather(ent_pad, rel_pad, e1, rel)
    ent_t = jnp.pad(ent_emb.T, ((0, 0), (0, N_PAD - NUM_ENT)))
    return pl.pallas_call(
        _score_body,
        grid=(B // B_BLK,),
        in_specs=[
            pl.BlockSpec((B_BLK, D_PAD), lambda i: (i, 0)),
            pl.BlockSpec((B_BLK, D_PAD), lambda i: (i, 0)),
            pl.BlockSpec((DIM, N_PAD), lambda i: (0, 0)),
        ],
        out_specs=pl.BlockSpec((B_BLK, NUM_ENT), lambda i: (i, 0)),
        out_shape=jax.ShapeDtypeStruct((B, NUM_ENT), jnp.float32),
    )(e1_rows, rel_rows, ent_t)


# speedup vs baseline: 1.3730x; 1.0096x over previous
import functools

import jax
import jax.numpy as jnp
from jax import lax
from jax.experimental import pallas as pl
from jax.experimental.pallas import tpu as pltpu
from jax.experimental.pallas import tpu_sc as plsc

B = 1024
NUM_ENT = 1000
DIM = 64
D_PAD = 128
N_PAD = 1024
B_BLK = 256
CH = 128
EPS = 1e-12


def _sc_gather(ent_emb, rel_emb, e1, rel):
    info = plsc.get_sparse_core_info()
    nw = info.num_cores * info.num_subcores
    b_per_w = B // nw
    mesh = plsc.VectorSubcoreMesh(core_axis_name="c", subcore_axis_name="s")

    @functools.partial(
        pl.kernel,
        mesh=mesh,
        out_type=[
            jax.ShapeDtypeStruct((B, D_PAD), jnp.float32),
            jax.ShapeDtypeStruct((B, D_PAD), jnp.float32),
        ],
        scratch_types=[
            pltpu.VMEM((b_per_w,), jnp.int32),
            pltpu.VMEM((b_per_w,), jnp.int32),
            pltpu.VMEM((b_per_w, D_PAD), jnp.float32),
            pltpu.VMEM((b_per_w, D_PAD), jnp.float32),
            pltpu.SemaphoreType.DMA,
            pltpu.SemaphoreType.DMA,
        ],
    )
    def gk(ent_hbm, rel_hbm, e1_hbm, ridx_hbm, oute_hbm, outr_hbm,
           idx1_v, idx2_v, rows1_v, rows2_v, sem1, sem2):
        wid = lax.axis_index("s") * info.num_cores + lax.axis_index("c")
        base = wid * b_per_w
        ci1 = pltpu.async_copy(e1_hbm.at[pl.ds(base, b_per_w)], idx1_v, sem1)
        ci2 = pltpu.async_copy(ridx_hbm.at[pl.ds(base, b_per_w)], idx2_v, sem2)
        ci1.wait()
        c1 = pltpu.async_copy(ent_hbm.at[idx1_v], rows1_v, sem1)
        ci2.wait()
        c2 = pltpu.async_copy(rel_hbm.at[idx2_v], rows2_v, sem2)
        c1.wait()
        co1 = pltpu.async_copy(rows1_v, oute_hbm.at[pl.ds(base, b_per_w)], sem1)
        c2.wait()
        co2 = pltpu.async_copy(rows2_v, outr_hbm.at[pl.ds(base, b_per_w)], sem2)
        co1.wait()
        co2.wait()

    return gk(ent_emb, rel_emb, e1, rel)


def _rnorm(x):
    n = jnp.sqrt(jnp.sum(x * x, axis=-1, keepdims=True))
    return x / jnp.maximum(n, EPS)


def _score_body(e1r_ref, relr_ref, entT_ref, out_ref):
    h = _rnorm(e1r_ref[:, :DIM]) + _rnorm(relr_ref[:, :DIM])

    ent_t = entT_ref[...]
    n = jnp.sqrt(jnp.sum(ent_t * ent_t, axis=0, keepdims=True))
    ent_tn = ent_t / jnp.maximum(n, EPS)
    lane = lax.broadcasted_iota(jnp.int32, (1, N_PAD), 1)
    sn = jnp.where(lane < NUM_ENT,
                   jnp.sum(ent_tn, axis=0, keepdims=True),
                   -jnp.inf)

    hb = [jnp.broadcast_to(h[:, d:d + 1], (B_BLK, CH)) for d in range(DIM)]

    chunks = []
    for c in range(N_PAD // CH):
        sl = ent_tn[:, c * CH:(c + 1) * CH]
        acc = jnp.minimum(hb[0], sl[0:1, :])
        for d in range(1, DIM):
            acc = acc + jnp.minimum(hb[d], sl[d:d + 1, :])
        chunks.append(sn[:, c * CH:(c + 1) * CH] - (acc + acc))
    dist = jnp.concatenate(chunks, axis=1)

    m = jnp.max(dist, axis=-1, keepdims=True)
    e = jnp.exp(dist - m)
    s = jnp.sum(e, axis=-1, keepdims=True)
    out_ref[...] = (e / s)[:, :NUM_ENT]


def kernel(e1, rel, X, A, ent_emb, rel_emb):
    del X, A
    e1 = e1.astype(jnp.int32)
    rel = rel.astype(jnp.int32)
    ent_pad = jnp.pad(ent_emb, ((0, 0), (0, D_PAD - DIM)))
    rel_pad = jnp.pad(rel_emb, ((0, 0), (0, D_PAD - DIM)))
    e1_rows, rel_rows = _sc_gather(ent_pad, rel_pad, e1, rel)
    ent_t = jnp.pad(ent_emb.T, ((0, 0), (0, N_PAD - NUM_ENT)))
    return pl.pallas_call(
        _score_body,
        grid=(B // B_BLK,),
        in_specs=[
            pl.BlockSpec((B_BLK, D_PAD), lambda i: (i, 0)),
            pl.BlockSpec((B_BLK, D_PAD), lambda i: (i, 0)),
            pl.BlockSpec((DIM, N_PAD), lambda i: (0, 0)),
        ],
        out_specs=pl.BlockSpec((B_BLK, NUM_ENT), lambda i: (i, 0)),
        out_shape=jax.ShapeDtypeStruct((B, NUM_ENT), jnp.float32),
    )(e1_rows, rel_rows, ent_t)


# speedup vs baseline: 1.3842x; 1.0081x over previous
import functools

import jax
import jax.numpy as jnp
from jax import lax
from jax.experimental import pallas as pl
from jax.experimental.pallas import tpu as pltpu
from jax.experimental.pallas import tpu_sc as plsc

B = 1024
NUM_ENT = 1000
DIM = 64
D_PAD = 128
N_PAD = 1024
B_BLK = 512
CH = 128
EPS = 1e-12


def _sc_gather(ent_emb, rel_emb, e1, rel):
    info = plsc.get_sparse_core_info()
    nw = info.num_cores * info.num_subcores
    b_per_w = B // nw
    mesh = plsc.VectorSubcoreMesh(core_axis_name="c", subcore_axis_name="s")

    @functools.partial(
        pl.kernel,
        mesh=mesh,
        out_type=[
            jax.ShapeDtypeStruct((B, D_PAD), jnp.float32),
            jax.ShapeDtypeStruct((B, D_PAD), jnp.float32),
        ],
        scratch_types=[
            pltpu.VMEM((b_per_w,), jnp.int32),
            pltpu.VMEM((b_per_w,), jnp.int32),
            pltpu.VMEM((b_per_w, D_PAD), jnp.float32),
            pltpu.VMEM((b_per_w, D_PAD), jnp.float32),
            pltpu.SemaphoreType.DMA,
            pltpu.SemaphoreType.DMA,
        ],
    )
    def gk(ent_hbm, rel_hbm, e1_hbm, ridx_hbm, oute_hbm, outr_hbm,
           idx1_v, idx2_v, rows1_v, rows2_v, sem1, sem2):
        wid = lax.axis_index("s") * info.num_cores + lax.axis_index("c")
        base = wid * b_per_w
        ci1 = pltpu.async_copy(e1_hbm.at[pl.ds(base, b_per_w)], idx1_v, sem1)
        ci2 = pltpu.async_copy(ridx_hbm.at[pl.ds(base, b_per_w)], idx2_v, sem2)
        ci1.wait()
        c1 = pltpu.async_copy(ent_hbm.at[idx1_v], rows1_v, sem1)
        ci2.wait()
        c2 = pltpu.async_copy(rel_hbm.at[idx2_v], rows2_v, sem2)
        c1.wait()
        co1 = pltpu.async_copy(rows1_v, oute_hbm.at[pl.ds(base, b_per_w)], sem1)
        c2.wait()
        co2 = pltpu.async_copy(rows2_v, outr_hbm.at[pl.ds(base, b_per_w)], sem2)
        co1.wait()
        co2.wait()

    return gk(ent_emb, rel_emb, e1, rel)


def _rnorm(x):
    n = jnp.sqrt(jnp.sum(x * x, axis=-1, keepdims=True))
    return x / jnp.maximum(n, EPS)


def _score_body(e1r_ref, relr_ref, entT_ref, out_ref):
    h = _rnorm(e1r_ref[:, :DIM]) + _rnorm(relr_ref[:, :DIM])

    ent_t = entT_ref[...]
    n = jnp.sqrt(jnp.sum(ent_t * ent_t, axis=0, keepdims=True))
    ent_tn = ent_t / jnp.maximum(n, EPS)
    lane = lax.broadcasted_iota(jnp.int32, (1, N_PAD), 1)
    sn = jnp.where(lane < NUM_ENT,
                   jnp.sum(ent_tn, axis=0, keepdims=True),
                   -jnp.inf)

    hb = [jnp.broadcast_to(h[:, d:d + 1], (B_BLK, CH)) for d in range(DIM)]

    chunks = []
    for c in range(N_PAD // CH):
        sl = ent_tn[:, c * CH:(c + 1) * CH]
        acc = jnp.minimum(hb[0], sl[0:1, :])
        for d in range(1, DIM):
            acc = acc + jnp.minimum(hb[d], sl[d:d + 1, :])
        chunks.append(sn[:, c * CH:(c + 1) * CH] - (acc + acc))
    dist = jnp.concatenate(chunks, axis=1)

    m = jnp.max(dist, axis=-1, keepdims=True)
    e = jnp.exp(dist - m)
    s = jnp.sum(e, axis=-1, keepdims=True)
    out_ref[...] = (e / s)[:, :NUM_ENT]


def kernel(e1, rel, X, A, ent_emb, rel_emb):
    del X, A
    e1 = e1.astype(jnp.int32)
    rel = rel.astype(jnp.int32)
    ent_pad = jnp.pad(ent_emb, ((0, 0), (0, D_PAD - DIM)))
    rel_pad = jnp.pad(rel_emb, ((0, 0), (0, D_PAD - DIM)))
    e1_rows, rel_rows = _sc_gather(ent_pad, rel_pad, e1, rel)
    ent_t = jnp.pad(ent_emb.T, ((0, 0), (0, N_PAD - NUM_ENT)))
    return pl.pallas_call(
        _score_body,
        grid=(B // B_BLK,),
        in_specs=[
            pl.BlockSpec((B_BLK, D_PAD), lambda i: (i, 0)),
            pl.BlockSpec((B_BLK, D_PAD), lambda i: (i, 0)),
            pl.BlockSpec((DIM, N_PAD), lambda i: (0, 0)),
        ],
        out_specs=pl.BlockSpec((B_BLK, NUM_ENT), lambda i: (i, 0)),
        out_shape=jax.ShapeDtypeStruct((B, NUM_ENT), jnp.float32),
    )(e1_rows, rel_rows, ent_t)
